# Initial kernel scaffold; baseline (speedup 1.0000x reference)
import jax
import jax.numpy as jnp
from jax.experimental import pallas as pl


def kernel(x, table, W1, b1, W2, b2):
    raise NotImplementedError("write your pallas kernel here")


# speedup vs baseline: 2.5127x; 2.5127x over previous
import functools

import jax
import jax.numpy as jnp
from jax import lax
from jax.experimental import pallas as pl
from jax.experimental.pallas import tpu as pltpu
from jax.experimental.pallas import tpu_sc as plsc

B = 16384
H = 50
D = 64
NC = 2
NS = 16
NW = NC * NS
BLK = 128
BAGS_PER_W = B // NW
NBLK = BAGS_PER_W // BLK
LANES = 16


def _sc_bag_sum(x_r, table):
  mesh = plsc.VectorSubcoreMesh(core_axis_name="c", subcore_axis_name="s")

  @functools.partial(
      pl.kernel,
      out_type=jax.ShapeDtypeStruct((B, D), jnp.float32),
      mesh=mesh,
      compiler_params=pltpu.CompilerParams(use_tc_tiling_on_sc=False),
      scratch_types=[
          pltpu.VMEM((H, BLK), jnp.int32),
          pltpu.VMEM((BLK, D), jnp.float32),
          pltpu.VMEM((BLK, D), jnp.float32),
          pltpu.VMEM((BLK, D), jnp.float32),
          pltpu.SemaphoreType.DMA,
          pltpu.SemaphoreType.DMA,
          pltpu.SemaphoreType.DMA,
      ],
  )
  def body(x_hbm, table_hbm, out_hbm, idx_v, rows0, rows1, acc, sem_a, sem0, sem1):
    wid = lax.axis_index("c") * NS + lax.axis_index("s")
    bufs = (rows0, rows1)
    sems = (sem0, sem1)

    def block_body(b, carry):
      pltpu.sync_copy(x_hbm.at[wid, b], idx_v)
      cp_acc = pltpu.async_copy(table_hbm.at[idx_v.at[0]], acc, sem_a)
      cps = {1: pltpu.async_copy(table_hbm.at[idx_v.at[1]], bufs[0], sems[0])}
      cp_acc.wait()
      for j in range(1, H):
        if j + 1 < H:
          cps[j + 1] = pltpu.async_copy(
              table_hbm.at[idx_v.at[j + 1]], bufs[j % 2], sems[j % 2])
        cps.pop(j).wait()
        buf = bufs[(j - 1) % 2]

        def acc_body(i, _):
          r = i * 4
          for rr in range(4):
            for d in range(4):
              sl = pl.ds(d * LANES, LANES)
              plsc.addupdate(acc.at[r + rr, sl], buf[r + rr, sl])
          return 0

        lax.fori_loop(0, BLK // 4, acc_body, 0)
      base = wid * BAGS_PER_W + b * BLK
      pltpu.sync_copy(acc, out_hbm.at[pl.ds(base, BLK)])
      return carry

    lax.fori_loop(0, NBLK, block_body, 0)

  return body(x_r, table)


def _tc_mlp_softmax(s, w1, b1, w2, b2):
  TB = 2048

  def body(s_ref, w1_ref, b1_ref, w2_ref, b2_ref, o_ref):
    h = jnp.dot(s_ref[...], w1_ref[...], preferred_element_type=jnp.float32)
    h = h + b1_ref[...]
    h = jnp.dot(h, w2_ref[...], preferred_element_type=jnp.float32)
    h = h + b2_ref[...]
    m = jnp.max(h, axis=1, keepdims=True)
    e = jnp.exp(h - m)
    o_ref[...] = e / jnp.sum(e, axis=1, keepdims=True)

  return pl.pallas_call(
      body,
      out_shape=jax.ShapeDtypeStruct((B, D), jnp.float32),
      grid=(B // TB,),
      in_specs=[
          pl.BlockSpec((TB, D), lambda i: (i, 0)),
          pl.BlockSpec((D, D), lambda i: (0, 0)),
          pl.BlockSpec((1, D), lambda i: (0, 0)),
          pl.BlockSpec((D, D), lambda i: (0, 0)),
          pl.BlockSpec((1, D), lambda i: (0, 0)),
      ],
      out_specs=pl.BlockSpec((TB, D), lambda i: (i, 0)),
  )(s, w1, b1, w2, b2)


@jax.jit
def kernel(x, table, W1, b1, W2, b2):
  x_r = x.reshape(NW, NBLK, BLK, H).transpose(0, 1, 3, 2)
  sums = _sc_bag_sum(x_r, table)
  w1s = W1.T / float(H)
  return _tc_mlp_softmax(sums, w1s, b1.reshape(1, D), W2.T, b2.reshape(1, D))
